# Initial kernel scaffold; baseline (speedup 1.0000x reference)
import jax
import jax.numpy as jnp
from jax.experimental import pallas as pl


def kernel(u_node, v_node, negative_nodes, W):
    raise NotImplementedError("write your pallas kernel here")


# speedup vs baseline: 2.0321x; 2.0321x over previous
import functools

import jax
import jax.numpy as jnp
from jax import lax
from jax.experimental import pallas as pl
from jax.experimental.pallas import tpu as pltpu
from jax.experimental.pallas import tpu_sc as plsc

B = 16384
D = 512
K = 20
ROWS_PER_SAMPLE = K + 2
NW = 32
NB = B // NW
G = 4
NG = NB // G
GROW = G * ROWS_PER_SAMPLE
DJ = D // 16


def _sc_scores(idx_packed, W):
    mesh = plsc.VectorSubcoreMesh(core_axis_name="c", subcore_axis_name="s")

    @functools.partial(
        pl.kernel,
        mesh=mesh,
        out_type=[
            jax.ShapeDtypeStruct((NW, NB // 8, 128), jnp.float32),
            jax.ShapeDtypeStruct((NW, NB // 8, 128), jnp.float32),
        ],
        scratch_types=[
            pltpu.VMEM((NG, GROW), jnp.int32),
            pltpu.VMEM((GROW, D), jnp.float32),
            pltpu.VMEM((NB // 8, 128), jnp.float32),
            pltpu.VMEM((NB // 8, 128), jnp.float32),
            pltpu.SemaphoreType.DMA,
        ],
    )
    def k(idx_hbm, w_hbm, pos_hbm, neg_hbm, idx_v, rows_v, pos_v, neg_v, sem):
        wid = lax.axis_index("s") * 2 + lax.axis_index("c")
        pltpu.sync_copy(idx_hbm.at[wid], idx_v)

        def group_body(g, carry):
            pltpu.async_copy(w_hbm.at[idx_v.at[g]], rows_v, sem).wait()

            def sample_body(s, carry2):
                r0 = s * ROWS_PER_SAMPLE
                u = [rows_v[r0, pl.ds(16 * j, 16)] for j in range(DJ)]
                pos = u[0] * rows_v[r0 + 1, pl.ds(0, 16)]
                for j in range(1, DJ):
                    pos = pos + u[j] * rows_v[r0 + 1, pl.ds(16 * j, 16)]

                def neg_body(kk, acc):
                    r = r0 + 2 + kk
                    a = acc
                    for j in range(DJ):
                        a = a + u[j] * rows_v[r, pl.ds(16 * j, 16)]
                    return a

                neg = lax.fori_loop(
                    0, K, neg_body, jnp.zeros((16,), jnp.float32))
                sg = g * G + s
                pos_v[sg // 8, pl.ds((sg % 8) * 16, 16)] = pos
                neg_v[sg // 8, pl.ds((sg % 8) * 16, 16)] = neg
                return carry2

            lax.fori_loop(0, G, sample_body, 0)
            return carry

        lax.fori_loop(0, NG, group_body, 0)
        pltpu.sync_copy(pos_v, pos_hbm.at[wid])
        pltpu.sync_copy(neg_v, neg_hbm.at[wid])

    return k(idx_packed, W)


def _tc_loss(pos_part, neg_part):
    def body(pos_ref, neg_ref, out_ref):
        pos = jnp.sum(pos_ref[...], axis=1)
        neg = -jnp.sum(neg_ref[...], axis=1)
        def logsig(x):
            return jnp.minimum(x, 0.0) - jnp.log1p(jnp.exp(-jnp.abs(x)))
        total = jnp.sum(logsig(pos) + logsig(neg))
        out_ref[...] = jnp.reshape(-total / B, (1, 1))

    return pl.pallas_call(
        body,
        out_shape=jax.ShapeDtypeStruct((1, 1), jnp.float32),
    )(pos_part, neg_part)


def kernel(u_node, v_node, negative_nodes, W):
    idx = jnp.concatenate(
        [u_node.astype(jnp.int32),
         v_node.astype(jnp.int32),
         negative_nodes.astype(jnp.int32)], axis=1)
    idx_packed = idx.reshape(NW, NG, GROW)
    pos_part, neg_part = _sc_scores(idx_packed, W)
    loss = _tc_loss(pos_part.reshape(B, 16), neg_part.reshape(B, 16))
    return loss.reshape(())


# speedup vs baseline: 3.3537x; 1.6504x over previous
import functools

import jax
import jax.numpy as jnp
from jax import lax
from jax.experimental import pallas as pl
from jax.experimental.pallas import tpu as pltpu
from jax.experimental.pallas import tpu_sc as plsc

B = 16384
D = 512
K = 20
ROWS_PER_SAMPLE = K + 2
NW = 32
NB = B // NW
G = 4
NG = NB // G
GROW = G * ROWS_PER_SAMPLE
DJ = D // 16


def _sc_scores(idx_packed, W):
    mesh = plsc.VectorSubcoreMesh(core_axis_name="c", subcore_axis_name="s")

    @functools.partial(
        pl.kernel,
        mesh=mesh,
        out_type=[
            jax.ShapeDtypeStruct((NW, NB // 8, 128), jnp.float32),
            jax.ShapeDtypeStruct((NW, NB // 8, 128), jnp.float32),
        ],
        scratch_types=[
            pltpu.VMEM((NG, GROW), jnp.int32),
            pltpu.VMEM((GROW, D), jnp.float32),
            pltpu.VMEM((GROW, D), jnp.float32),
            pltpu.VMEM((NB // 8, 128), jnp.float32),
            pltpu.VMEM((NB // 8, 128), jnp.float32),
            pltpu.SemaphoreType.DMA,
            pltpu.SemaphoreType.DMA,
        ],
    )
    def k(idx_hbm, w_hbm, pos_hbm, neg_hbm,
          idx_v, rows0, rows1, pos_v, neg_v, sem0, sem1):
        wid = lax.axis_index("s") * 2 + lax.axis_index("c")
        pltpu.sync_copy(idx_hbm.at[wid], idx_v)

        def compute(g, rows_v):
            def sample_body(s, carry2):
                r0 = s * ROWS_PER_SAMPLE
                u = [rows_v[r0, pl.ds(16 * j, 16)] for j in range(DJ)]
                pos = u[0] * rows_v[r0 + 1, pl.ds(0, 16)]
                for j in range(1, DJ):
                    pos = pos + u[j] * rows_v[r0 + 1, pl.ds(16 * j, 16)]

                def neg_body(kk, acc):
                    r = r0 + 2 + kk
                    a = acc
                    for j in range(DJ):
                        a = a + u[j] * rows_v[r, pl.ds(16 * j, 16)]
                    return a

                neg = lax.fori_loop(
                    0, K, neg_body, jnp.zeros((16,), jnp.float32))
                sg = g * G + s
                pos_v[sg // 8, pl.ds((sg % 8) * 16, 16)] = pos
                neg_v[sg // 8, pl.ds((sg % 8) * 16, 16)] = neg
                return carry2

            lax.fori_loop(0, G, sample_body, 0)

        pltpu.async_copy(w_hbm.at[idx_v.at[0]], rows0, sem0)

        def pair_body(i, carry):
            g = 2 * i
            pltpu.make_async_copy(w_hbm.at[idx_v.at[g]], rows0, sem0).wait()
            pltpu.async_copy(w_hbm.at[idx_v.at[g + 1]], rows1, sem1)
            compute(g, rows0)
            pltpu.make_async_copy(w_hbm.at[idx_v.at[g + 1]], rows1, sem1).wait()

            @pl.when(i < NG // 2 - 1)
            def _():
                pltpu.async_copy(w_hbm.at[idx_v.at[g + 2]], rows0, sem0)

            compute(g + 1, rows1)
            return carry

        lax.fori_loop(0, NG // 2, pair_body, 0)
        pltpu.sync_copy(pos_v, pos_hbm.at[wid])
        pltpu.sync_copy(neg_v, neg_hbm.at[wid])

    return k(idx_packed, W)


def _tc_loss(pos_part, neg_part):
    def body(pos_ref, neg_ref, out_ref):
        pos = jnp.sum(pos_ref[...], axis=1)
        neg = -jnp.sum(neg_ref[...], axis=1)
        def logsig(x):
            return jnp.minimum(x, 0.0) - jnp.log1p(jnp.exp(-jnp.abs(x)))
        total = jnp.sum(logsig(pos) + logsig(neg))
        out_ref[...] = jnp.reshape(-total / B, (1, 1))

    return pl.pallas_call(
        body,
        out_shape=jax.ShapeDtypeStruct((1, 1), jnp.float32),
    )(pos_part, neg_part)


def kernel(u_node, v_node, negative_nodes, W):
    idx = jnp.concatenate(
        [u_node.astype(jnp.int32),
         v_node.astype(jnp.int32),
         negative_nodes.astype(jnp.int32)], axis=1)
    idx_packed = idx.reshape(NW, NG, GROW)
    pos_part, neg_part = _sc_scores(idx_packed, W)
    loss = _tc_loss(pos_part.reshape(B, 16), neg_part.reshape(B, 16))
    return loss.reshape(())
